# Initial kernel scaffold; baseline (speedup 1.0000x reference)
import jax
import jax.numpy as jnp
from jax.experimental import pallas as pl


def kernel(features_0, features_1, features_2, features_target, edge_index, meta_path_0, meta_path_1, params):
    raise NotImplementedError("write your pallas kernel here")


# speedup vs baseline: 1.0007x; 1.0007x over previous
import functools

import jax
import jax.numpy as jnp
from jax.experimental import pallas as pl

N0, N0_REAL, N1, N2 = 3000, 2250, 3500, 3500
N_TOTAL = N0 + N1 + N2
N_REAL = N0_REAL + N1 + N2
IN_DIM, HID, NHID, NHEADS, NMP, NCLS = 128, 64, 32, 4, 2, 8
E_EDGES = 320000
FC_HID = int(2.0 / 3.0 * NHID) + int(2.0 / 3.0 * NCLS)

_f32 = jnp.float32


def _elu(x):
    return jnp.where(x > 0, x, jnp.exp(jnp.minimum(x, 0.0)) - 1.0)


def _lrelu(x):
    return jnp.where(x > 0, x, 0.2 * x)


def _mm_body(x_ref, w_ref, b_ref, o_ref, *, act):
    y = jnp.dot(x_ref[...], w_ref[...], preferred_element_type=_f32)
    if b_ref is not None:
        y = y + b_ref[...]
    o_ref[...] = act(y) if act is not None else y


def _mm_nb_body(x_ref, w_ref, o_ref, *, act):
    _mm_body(x_ref, w_ref, None, o_ref, act=act)


def _mm(x, w, b=None, act=None, tr=512):
    n, k = x.shape
    m = w.shape[1]
    n_pad = ((n + tr - 1) // tr) * tr
    if n_pad != n:
        x = jnp.pad(x, ((0, n_pad - n), (0, 0)))
    specs = [
        pl.BlockSpec((tr, k), lambda i: (i, 0)),
        pl.BlockSpec((k, m), lambda i: (0, 0)),
    ]
    args = [x, w]
    if b is not None:
        specs.append(pl.BlockSpec((1, m), lambda i: (0, 0)))
        args.append(b.reshape(1, m))
        body = functools.partial(_mm_body, act=act)
    else:
        body = functools.partial(_mm_nb_body, act=act)
    out = pl.pallas_call(
        body,
        grid=(n_pad // tr,),
        in_specs=specs,
        out_specs=pl.BlockSpec((tr, m), lambda i: (i, 0)),
        out_shape=jax.ShapeDtypeStruct((n_pad, m), _f32),
    )(*args)
    return out[:n] if n_pad != n else out


def _mm_heads_body(x_ref, *w_refs, o_ref, act, nh):
    x = x_ref[...]
    ys = [jnp.dot(x, w_refs[j][...].T, preferred_element_type=_f32)
          for j in range(nh)]
    y = jnp.concatenate(ys, axis=1)
    o_ref[...] = act(y) if act is not None else y


def _mm_heads(x, ws, act=None, tr=512):
    n, k = x.shape
    nh = len(ws)
    m = sum(w.shape[0] for w in ws)
    n_pad = ((n + tr - 1) // tr) * tr
    if n_pad != n:
        x = jnp.pad(x, ((0, n_pad - n), (0, 0)))
    specs = [pl.BlockSpec((tr, k), lambda i: (i, 0))]
    for w in ws:
        specs.append(pl.BlockSpec(w.shape, lambda i: (0, 0)))

    def body(x_ref, *w_refs_and_o):
        w_refs, o_ref = w_refs_and_o[:-1], w_refs_and_o[-1]
        _mm_heads_body(x_ref, *w_refs, o_ref=o_ref, act=act, nh=nh)

    out = pl.pallas_call(
        body,
        grid=(n_pad // tr,),
        in_specs=specs,
        out_specs=pl.BlockSpec((tr, m), lambda i: (i, 0)),
        out_shape=jax.ShapeDtypeStruct((n_pad, m), _f32),
    )(x, *ws)
    return out[:n] if n_pad != n else out


def _pred_body(zc_ref, zf_ref, mp_ref, pred_ref, fin_ref, *, tr):
    i = pl.program_id(0)
    p = jax.nn.sigmoid(jax.lax.dot_general(
        zc_ref[...], zf_ref[...], (((1,), (1,)), ((), ())),
        preferred_element_type=_f32))
    rows = i * tr + jax.lax.broadcasted_iota(jnp.int32, (tr, 1), 0)
    cols = jax.lax.broadcasted_iota(jnp.int32, (1, N0), 1)
    mask = (rows >= N0_REAL) | (cols >= N0_REAL)
    pred_ref[...] = p
    fin_ref[...] = jnp.where(mask, p, mp_ref[...])


def _meta_pred_m(zc, mp_real, tr=120):
    return pl.pallas_call(
        functools.partial(_pred_body, tr=tr),
        grid=(N0 // tr,),
        in_specs=[
            pl.BlockSpec((tr, NHID), lambda i: (i, 0)),
            pl.BlockSpec((N0, NHID), lambda i: (0, 0)),
            pl.BlockSpec((tr, N0), lambda i: (i, 0)),
        ],
        out_specs=[
            pl.BlockSpec((tr, N0), lambda i: (i, 0)),
            pl.BlockSpec((tr, N0), lambda i: (i, 0)),
        ],
        out_shape=[
            jax.ShapeDtypeStruct((N0, N0), _f32),
            jax.ShapeDtypeStruct((N0, N0), _f32),
        ],
    )(zc, zc, mp_real)


def _dense_attn_body(adj_ref, ft_ref, ff_ref, *rest, nh):
    a1_refs = rest[:nh]
    a2_refs = rest[nh:2 * nh]
    b_refs = rest[2 * nh:4 * nh]
    o_ref = rest[-1]
    adj = adj_ref[...]
    ft = ft_ref[...]
    ff = ff_ref[...]
    outs = []
    for j in range(nh):
        fj_t = ft[:, j * NHID:(j + 1) * NHID]
        fj_f = ff[:, j * NHID:(j + 1) * NHID]
        f1 = jnp.dot(fj_t, a1_refs[j][...].T, preferred_element_type=_f32)
        f2 = jax.lax.dot_general(a2_refs[j][...], fj_f,
                                 (((1,), (1,)), ((), ())),
                                 preferred_element_type=_f32)
        logits = _lrelu(f1 + f2)
        mx = jnp.max(logits, axis=1, keepdims=True)
        e = jnp.exp(logits - mx)
        sm = e / jnp.sum(e, axis=1, keepdims=True)
        c = sm * adj
        outs.append(jnp.dot(c, fj_f, preferred_element_type=_f32))
    y = jnp.concatenate(outs, axis=1) if nh > 1 else outs[0]
    bias = jnp.concatenate([b_refs[j][...] for j in range(nh)], axis=1) \
        if nh > 1 else b_refs[0][...]
    o_ref[...] = _elu(y + bias)


def _dense_attn_m(adj, fts, a1s, a2s, bs, nh, tr=120):
    w = nh * NHID
    specs = [
        pl.BlockSpec((tr, N0), lambda i: (i, 0)),
        pl.BlockSpec((tr, w), lambda i: (i, 0)),
        pl.BlockSpec((N0, w), lambda i: (0, 0)),
    ]
    for _ in range(3 * nh):
        specs.append(pl.BlockSpec((1, NHID), lambda i: (0, 0)))

    def body(adj_ref, ft_ref, ff_ref, *rest):
        _dense_attn_body(adj_ref, ft_ref, ff_ref, *rest, nh=nh)

    return pl.pallas_call(
        body,
        grid=(N0 // tr,),
        in_specs=specs,
        out_specs=pl.BlockSpec((tr, w), lambda i: (i, 0)),
        out_shape=jax.ShapeDtypeStruct((N0, w), _f32),
    )(adj, fts, fts, *a1s, *a2s, *bs)


def _sp_attn_multi(fts, f1, f2, bias, src, dst, nh):
    e = _lrelu(f1[src] + f2[dst])
    mseg = jax.ops.segment_max(e, dst, num_segments=N_REAL)
    a = jnp.exp(e - mseg[dst])
    denom = jax.ops.segment_sum(a, dst, num_segments=N_REAL)
    a = a / (denom[dst] + 1e-9)
    g = fts.reshape(N_REAL, nh, NHID)[src]
    out = jax.ops.segment_sum(a[:, :, None] * g, dst, num_segments=N_REAL)
    return _elu(out.reshape(N_REAL, nh * NHID) + bias)


def kernel(features_0, features_1, features_2, features_target, edge_index,
           meta_path_0, meta_path_1, params):
    p = params
    src, dst = edge_index[0], edge_index[1]

    h0p = _mm(features_0, p["W_fc0"].T, p["b_fc0"], _elu)
    h1p = _mm(features_1, p["W_fc1"].T, p["b_fc1"], _elu)
    _DBG_STAGE = 0
    h2p = _mm(features_2, p["W_fc2"].T, p["b_fc2"], _elu)

    preds, final_mps = [], []
    for m in range(NMP):
        zc = _mm(features_target, p["W_mp%d" % m].T, p["b_mp%d" % m])
        pr, fi = _meta_pred_m(zc, (meta_path_0, meta_path_1)[m])
        preds.append(pr)
        final_mps.append(fi)

    if _DBG_STAGE == 2:
        acc = jnp.float32(0)
        for m in range(NMP):
            hs = [p["m%d_%d" % (m, j)] for j in range(NHEADS)]
            ftsd = _mm_heads(h0p, [hp["Wf"] for hp in hs])
            sup = _dense_attn_m(final_mps[m], ftsd,
                                [hp["a1"].reshape(1, NHID) for hp in hs],
                                [hp["a2"].reshape(1, NHID) for hp in hs],
                                [hp["b"].reshape(1, NHID) for hp in hs],
                                NHEADS)
            mo = p["mo%d" % m]
            fts_mo = _mm(sup, mo["Wf"].T)
            so = _dense_attn_m(final_mps[m], fts_mo,
                               [mo["a1"].reshape(1, NHID)],
                               [mo["a2"].reshape(1, NHID)],
                               [mo["b"].reshape(1, NHID)], 1)
            acc = acc + jnp.mean(so)
        z = (jnp.mean(h1p) + jnp.mean(h2p)) * 0 + acc * 0
        return (preds[0] + z, preds[1], jnp.zeros((N0, NCLS), _f32),
                jnp.float32(0))
    if _DBG_STAGE == 1:
        z = (jnp.mean(h0p) + jnp.mean(h1p) + jnp.mean(h2p)
             + jnp.mean(final_mps[0]) + jnp.mean(final_mps[1])) * 0
        return (preds[0] + z, preds[1], jnp.zeros((N0, NCLS), _f32),
                jnp.float32(0))

    wf_sp = jnp.concatenate([p["sp%d" % j]["Wf"] for j in range(NHEADS)], 0)
    h_real = jnp.concatenate([
        _elu(features_0[:N0_REAL] @ p["W_fc0"].T + p["b_fc0"]),
        _elu(features_1 @ p["W_fc1"].T + p["b_fc1"]),
        _elu(features_2 @ p["W_fc2"].T + p["b_fc2"]),
    ], axis=0)
    fts_sp = h_real @ wf_sp.T
    a1_sp = jnp.stack([p["sp%d" % j]["a1"] for j in range(NHEADS)])
    a2_sp = jnp.stack([p["sp%d" % j]["a2"] for j in range(NHEADS)])
    fts_sp4 = fts_sp.reshape(N_REAL, NHEADS, NHID)
    f1_sp = jnp.einsum("nhc,hc->nh", fts_sp4, a1_sp)
    f2_sp = jnp.einsum("nhc,hc->nh", fts_sp4, a2_sp)
    b_sp = jnp.concatenate([p["sp%d" % j]["b"] for j in range(NHEADS)])
    h1cat = _sp_attn_multi(fts_sp, f1_sp, f2_sp, b_sp, src, dst, NHEADS)

    po = p["sp_out"]
    fts_o = h1cat @ po["Wf"].T
    f1_o = (fts_o @ po["a1"])[:, None]
    f2_o = (fts_o @ po["a2"])[:, None]
    h1_sp = _sp_attn_multi(fts_o, f1_o, f2_o, po["b"], src, dst, 1)

    h1_pred = h0p @ p["W_emb"].T + p["b_emb"]
    loss_embed = jnp.mean((h1_pred[:N0_REAL] - h1_sp[:N0_REAL]) ** 2)
    h1_full = jnp.concatenate([h1_sp[:N0_REAL], h1_pred[N0_REAL:]], axis=0)

    sup_o = []
    for m in range(NMP):
        hs = [p["m%d_%d" % (m, j)] for j in range(NHEADS)]
        ftsd = _mm_heads(h0p, [hp["Wf"] for hp in hs])
        sup = _dense_attn_m(final_mps[m], ftsd,
                            [hp["a1"].reshape(1, NHID) for hp in hs],
                            [hp["a2"].reshape(1, NHID) for hp in hs],
                            [hp["b"].reshape(1, NHID) for hp in hs], NHEADS)
        mo = p["mo%d" % m]
        fts_mo = _mm(sup, mo["Wf"].T)
        sup_o.append(_dense_attn_m(final_mps[m], fts_mo,
                                   [mo["a1"].reshape(1, NHID)],
                                   [mo["a2"].reshape(1, NHID)],
                                   [mo["b"].reshape(1, NHID)], 1))

    w_red = p["W_red"][0]
    fin = _elu(h1_full * w_red[0] + sup_o[0] * w_red[1] + sup_o[1] * w_red[2]
               + p["b_red"][0])
    out = _elu(fin @ p["W_o1"].T + p["b_o1"])
    out = jax.nn.log_softmax(out @ p["W_o2"].T + p["b_o2"], axis=1)
    return (preds[0], preds[1], out, loss_embed)


# speedup vs baseline: 1.0839x; 1.0832x over previous
import functools

import jax
import jax.numpy as jnp
from jax.experimental import pallas as pl

N0, N0_REAL, N1, N2 = 3000, 2250, 3500, 3500
N_TOTAL = N0 + N1 + N2
N_REAL = N0_REAL + N1 + N2
IN_DIM, HID, NHID, NHEADS, NMP, NCLS = 128, 64, 32, 4, 2, 8
E_EDGES = 320000
FC_HID = int(2.0 / 3.0 * NHID) + int(2.0 / 3.0 * NCLS)

_f32 = jnp.float32


def _elu(x):
    return jnp.where(x > 0, x, jnp.exp(jnp.minimum(x, 0.0)) - 1.0)


def _lrelu(x):
    return jnp.where(x > 0, x, 0.2 * x)


def _mm_body(x_ref, w_ref, b_ref, o_ref, *, act):
    y = jnp.dot(x_ref[...], w_ref[...], preferred_element_type=_f32)
    if b_ref is not None:
        y = y + b_ref[...]
    o_ref[...] = act(y) if act is not None else y


def _mm_nb_body(x_ref, w_ref, o_ref, *, act):
    _mm_body(x_ref, w_ref, None, o_ref, act=act)


def _mm(x, w, b=None, act=None, tr=512):
    n, k = x.shape
    m = w.shape[1]
    n_pad = ((n + tr - 1) // tr) * tr
    if n_pad != n:
        x = jnp.pad(x, ((0, n_pad - n), (0, 0)))
    specs = [
        pl.BlockSpec((tr, k), lambda i: (i, 0)),
        pl.BlockSpec((k, m), lambda i: (0, 0)),
    ]
    args = [x, w]
    if b is not None:
        specs.append(pl.BlockSpec((1, m), lambda i: (0, 0)))
        args.append(b.reshape(1, m))
        body = functools.partial(_mm_body, act=act)
    else:
        body = functools.partial(_mm_nb_body, act=act)
    out = pl.pallas_call(
        body,
        grid=(n_pad // tr,),
        in_specs=specs,
        out_specs=pl.BlockSpec((tr, m), lambda i: (i, 0)),
        out_shape=jax.ShapeDtypeStruct((n_pad, m), _f32),
    )(*args)
    return out[:n] if n_pad != n else out


def _mm_heads_body(x_ref, *w_refs, o_ref, act, nh):
    x = x_ref[...]
    ys = [jnp.dot(x, w_refs[j][...].T, preferred_element_type=_f32)
          for j in range(nh)]
    y = jnp.concatenate(ys, axis=1)
    o_ref[...] = act(y) if act is not None else y


def _mm_heads(x, ws, act=None, tr=512):
    n, k = x.shape
    nh = len(ws)
    m = sum(w.shape[0] for w in ws)
    n_pad = ((n + tr - 1) // tr) * tr
    if n_pad != n:
        x = jnp.pad(x, ((0, n_pad - n), (0, 0)))
    specs = [pl.BlockSpec((tr, k), lambda i: (i, 0))]
    for w in ws:
        specs.append(pl.BlockSpec(w.shape, lambda i: (0, 0)))

    def body(x_ref, *w_refs_and_o):
        w_refs, o_ref = w_refs_and_o[:-1], w_refs_and_o[-1]
        _mm_heads_body(x_ref, *w_refs, o_ref=o_ref, act=act, nh=nh)

    out = pl.pallas_call(
        body,
        grid=(n_pad // tr,),
        in_specs=specs,
        out_specs=pl.BlockSpec((tr, m), lambda i: (i, 0)),
        out_shape=jax.ShapeDtypeStruct((n_pad, m), _f32),
    )(x, *ws)
    return out[:n] if n_pad != n else out


def _pred_body(zc_ref, zf_ref, mp_ref, pred_ref, fin_ref, *, tr):
    i = pl.program_id(0)
    p = jax.nn.sigmoid(jax.lax.dot_general(
        zc_ref[...], zf_ref[...], (((1,), (1,)), ((), ())),
        preferred_element_type=_f32))
    rows = i * tr + jax.lax.broadcasted_iota(jnp.int32, (tr, 1), 0)
    cols = jax.lax.broadcasted_iota(jnp.int32, (1, N0), 1)
    mask = (rows >= N0_REAL) | (cols >= N0_REAL)
    pred_ref[...] = p
    fin_ref[...] = jnp.where(mask, p, mp_ref[...])


def _meta_pred_m(zc, mp_real, tr=120):
    return pl.pallas_call(
        functools.partial(_pred_body, tr=tr),
        grid=(N0 // tr,),
        in_specs=[
            pl.BlockSpec((tr, NHID), lambda i: (i, 0)),
            pl.BlockSpec((N0, NHID), lambda i: (0, 0)),
            pl.BlockSpec((tr, N0), lambda i: (i, 0)),
        ],
        out_specs=[
            pl.BlockSpec((tr, N0), lambda i: (i, 0)),
            pl.BlockSpec((tr, N0), lambda i: (i, 0)),
        ],
        out_shape=[
            jax.ShapeDtypeStruct((N0, N0), _f32),
            jax.ShapeDtypeStruct((N0, N0), _f32),
        ],
    )(zc, zc, mp_real)


def _dense_attn_body(adj_ref, ft_ref, ff_ref, *rest, nh):
    a1_refs = rest[:nh]
    a2_refs = rest[nh:2 * nh]
    b_refs = rest[2 * nh:4 * nh]
    o_ref = rest[-1]
    adj = adj_ref[...]
    ft = ft_ref[...]
    ff = ff_ref[...]
    outs = []
    for j in range(nh):
        fj_t = ft[:, j * NHID:(j + 1) * NHID]
        fj_f = ff[:, j * NHID:(j + 1) * NHID]
        f1 = jnp.dot(fj_t, a1_refs[j][...].T, preferred_element_type=_f32)
        f2 = jax.lax.dot_general(a2_refs[j][...], fj_f,
                                 (((1,), (1,)), ((), ())),
                                 preferred_element_type=_f32)
        logits = _lrelu(f1 + f2)
        mx = jnp.max(logits, axis=1, keepdims=True)
        e = jnp.exp(logits - mx)
        sm = e / jnp.sum(e, axis=1, keepdims=True)
        c = sm * adj
        outs.append(jnp.dot(c, fj_f, preferred_element_type=_f32))
    y = jnp.concatenate(outs, axis=1) if nh > 1 else outs[0]
    bias = jnp.concatenate([b_refs[j][...] for j in range(nh)], axis=1) \
        if nh > 1 else b_refs[0][...]
    o_ref[...] = _elu(y + bias)


def _dense_attn_m(adj, fts, a1s, a2s, bs, nh, tr=120):
    w = nh * NHID
    specs = [
        pl.BlockSpec((tr, N0), lambda i: (i, 0)),
        pl.BlockSpec((tr, w), lambda i: (i, 0)),
        pl.BlockSpec((N0, w), lambda i: (0, 0)),
    ]
    for _ in range(3 * nh):
        specs.append(pl.BlockSpec((1, NHID), lambda i: (0, 0)))

    def body(adj_ref, ft_ref, ff_ref, *rest):
        _dense_attn_body(adj_ref, ft_ref, ff_ref, *rest, nh=nh)

    return pl.pallas_call(
        body,
        grid=(N0 // tr,),
        in_specs=specs,
        out_specs=pl.BlockSpec((tr, w), lambda i: (i, 0)),
        out_shape=jax.ShapeDtypeStruct((N0, w), _f32),
    )(adj, fts, fts, *a1s, *a2s, *bs)


def _sp_attn_multi(fts, f1, f2, bias, src, dst, nh):
    mbound = _lrelu(jnp.max(f1, axis=0) + jnp.max(f2, axis=0))
    e = _lrelu(f1[src] + f2[dst])
    a = jnp.exp(e - mbound[None, :])
    denom = jax.ops.segment_sum(a, dst, num_segments=N_REAL)
    w = a / (denom[dst] + 1e-30)
    g = fts.reshape(N_REAL, nh, NHID)[src]
    out = jax.ops.segment_sum(w[:, :, None] * g, dst, num_segments=N_REAL)
    return _elu(out.reshape(N_REAL, nh * NHID) + bias)


def kernel(features_0, features_1, features_2, features_target, edge_index,
           meta_path_0, meta_path_1, params):
    p = params
    src, dst = edge_index[0], edge_index[1]

    h0p = _mm(features_0, p["W_fc0"].T, p["b_fc0"], _elu)
    h1p = _mm(features_1, p["W_fc1"].T, p["b_fc1"], _elu)
    h2p = _mm(features_2, p["W_fc2"].T, p["b_fc2"], _elu)

    preds, final_mps = [], []
    for m in range(NMP):
        zc = _mm(features_target, p["W_mp%d" % m].T, p["b_mp%d" % m])
        pr, fi = _meta_pred_m(zc, (meta_path_0, meta_path_1)[m])
        preds.append(pr)
        final_mps.append(fi)

    wf_sp = jnp.concatenate([p["sp%d" % j]["Wf"] for j in range(NHEADS)], 0)
    h_real = jnp.concatenate([
        _elu(features_0[:N0_REAL] @ p["W_fc0"].T + p["b_fc0"]),
        _elu(features_1 @ p["W_fc1"].T + p["b_fc1"]),
        _elu(features_2 @ p["W_fc2"].T + p["b_fc2"]),
    ], axis=0)
    fts_sp = h_real @ wf_sp.T
    a1_sp = jnp.stack([p["sp%d" % j]["a1"] for j in range(NHEADS)])
    a2_sp = jnp.stack([p["sp%d" % j]["a2"] for j in range(NHEADS)])
    fts_sp4 = fts_sp.reshape(N_REAL, NHEADS, NHID)
    f1_sp = jnp.einsum("nhc,hc->nh", fts_sp4, a1_sp)
    f2_sp = jnp.einsum("nhc,hc->nh", fts_sp4, a2_sp)
    b_sp = jnp.concatenate([p["sp%d" % j]["b"] for j in range(NHEADS)])
    h1cat = _sp_attn_multi(fts_sp, f1_sp, f2_sp, b_sp, src, dst, NHEADS)

    po = p["sp_out"]
    fts_o = h1cat @ po["Wf"].T
    f1_o = (fts_o @ po["a1"])[:, None]
    f2_o = (fts_o @ po["a2"])[:, None]
    h1_sp = _sp_attn_multi(fts_o, f1_o, f2_o, po["b"], src, dst, 1)

    h1_pred = h0p @ p["W_emb"].T + p["b_emb"]
    loss_embed = jnp.mean((h1_pred[:N0_REAL] - h1_sp[:N0_REAL]) ** 2)
    h1_full = jnp.concatenate([h1_sp[:N0_REAL], h1_pred[N0_REAL:]], axis=0)

    sup_o = []
    for m in range(NMP):
        hs = [p["m%d_%d" % (m, j)] for j in range(NHEADS)]
        ftsd = _mm_heads(h0p, [hp["Wf"] for hp in hs])
        sup = _dense_attn_m(final_mps[m], ftsd,
                            [hp["a1"].reshape(1, NHID) for hp in hs],
                            [hp["a2"].reshape(1, NHID) for hp in hs],
                            [hp["b"].reshape(1, NHID) for hp in hs], NHEADS)
        mo = p["mo%d" % m]
        fts_mo = _mm(sup, mo["Wf"].T)
        sup_o.append(_dense_attn_m(final_mps[m], fts_mo,
                                   [mo["a1"].reshape(1, NHID)],
                                   [mo["a2"].reshape(1, NHID)],
                                   [mo["b"].reshape(1, NHID)], 1))

    w_red = p["W_red"][0]
    fin = _elu(h1_full * w_red[0] + sup_o[0] * w_red[1] + sup_o[1] * w_red[2]
               + p["b_red"][0])
    out = _elu(fin @ p["W_o1"].T + p["b_o1"])
    out = jax.nn.log_softmax(out @ p["W_o2"].T + p["b_o2"], axis=1)
    return (preds[0], preds[1], out, loss_embed)
